# baseline (speedup 1.0000x reference)
import functools

import jax
import jax.numpy as jnp
from jax import lax
from jax.experimental import pallas as pl
from jax.experimental.pallas import tpu as pltpu
from jax.experimental.pallas import tpu_sc as plsc

_B, _F, _V, _E, _NUM = 4096, 26, 100000, 64, 13
_D = _F * _E + _E
_NC, _NS = 2, 16
_NW = _NC * _NS
_RPW = _B * _F // _NW
_J = _RPW // 128


def _make_sc_gather():
    mesh = plsc.VectorSubcoreMesh(
        core_axis_name="c", subcore_axis_name="s",
        num_cores=_NC, num_subcores=_NS)

    @functools.partial(
        pl.kernel,
        out_type=jax.ShapeDtypeStruct((_NW, _J, 128, _E), jnp.float32),
        mesh=mesh,
        scratch_types=[
            pltpu.VMEM((_J, 128), jnp.int32),
            pltpu.VMEM((128, _E), jnp.float32),
            pltpu.SemaphoreType.DMA,
        ],
        compiler_params=pltpu.CompilerParams(use_tc_tiling_on_sc=False),
    )
    def sc_gather(idx_hbm, tab_hbm, out_hbm, idx_v, rows_v, sem):
        wid = lax.axis_index("s") * _NC + lax.axis_index("c")
        pltpu.sync_copy(idx_hbm.at[wid], idx_v)

        def body(j, carry):
            pltpu.async_copy(tab_hbm.at[idx_v.at[j]], rows_v, sem).wait()
            pltpu.sync_copy(rows_v, out_hbm.at[wid, j])
            return carry

        lax.fori_loop(0, _J, body, 0, unroll=False)

    return sc_gather


_SC_GATHER_CACHE = []


def _sc_gather_fn():
    if not _SC_GATHER_CACHE:
        _SC_GATHER_CACHE.append(_make_sc_gather())
    return _SC_GATHER_CACHE[0]


def _tc_dense_body(cat_ref, xn_ref, wn_ref, bn_ref, wf_ref, bf_ref, out_ref):
    num_emb = lax.dot_general(
        xn_ref[...], wn_ref[...], (((1,), (1,)), ((), ())),
        preferred_element_type=jnp.float32) + bn_ref[...]
    wf = wf_ref[...]
    o = lax.dot_general(
        cat_ref[...], wf[:, : _F * _E], (((1,), (1,)), ((), ())),
        preferred_element_type=jnp.float32)
    o = o + lax.dot_general(
        num_emb, wf[:, _F * _E:], (((1,), (1,)), ((), ())),
        preferred_element_type=jnp.float32)
    out_ref[...] = o + bf_ref[...]


def _tc_dense(cat_emb, X_num, W_num, b_num2, W_final, b_final2):
    BB = 1024
    return pl.pallas_call(
        _tc_dense_body,
        grid=(_B // BB,),
        in_specs=[
            pl.BlockSpec((BB, _F * _E), lambda i: (i, 0)),
            pl.BlockSpec((BB, _NUM), lambda i: (i, 0)),
            pl.BlockSpec((_E, _NUM), lambda i: (0, 0)),
            pl.BlockSpec((1, _E), lambda i: (0, 0)),
            pl.BlockSpec((_E, _D), lambda i: (0, 0)),
            pl.BlockSpec((1, _E), lambda i: (0, 0)),
        ],
        out_specs=pl.BlockSpec((BB, _E), lambda i: (i, 0)),
        out_shape=jax.ShapeDtypeStruct((_B, _E), jnp.float32),
    )(cat_emb, X_num, W_num, b_num2, W_final, b_final2)


def kernel(X_cat, X_num, tables, W_num, b_num, W_final, b_final):
    flat_idx = (X_cat.astype(jnp.int32)
                + (jnp.arange(_F, dtype=jnp.int32) * _V)[None, :])
    flat_idx = flat_idx.reshape(_NW, _J, 128)
    flat_tab = tables.reshape(_F * _V, _E)
    gathered = _sc_gather_fn()(flat_idx, flat_tab)
    cat_emb = gathered.reshape(_B, _F * _E)
    return _tc_dense(cat_emb, X_num, W_num,
                     b_num.reshape(1, _E), W_final, b_final.reshape(1, _E))
